# baseline (speedup 1.0000x reference)
import functools

import jax
import jax.numpy as jnp
from jax import lax
from jax.experimental import pallas as pl
from jax.experimental.pallas import tpu as pltpu
from jax.experimental.pallas import tpu_sc as plsc

D = 128
NMIN = 60
NHOUR = 24
NC = 2
NS = 16
NW = NC * NS
L = 16
CHUNK = 128
NFEAT = 5
MIN_CH = 3
HOUR_CH = 2


def _idx_kernel(xt_ref, minute_ref, hour_ref, idx_ref, c_ref):
    @pl.when(pl.program_id(0) == 0)
    def _():
        c_ref[...] = minute_ref[...][:, None, :] + hour_ref[...][None, :, :]

    m = (xt_ref[MIN_CH] * 59.0).astype(jnp.int32)
    h = (xt_ref[HOUR_CH] * 23.0).astype(jnp.int32)
    idx_ref[0] = m * NHOUR + h


def _token_idx(x_mark, minute_embed, hour_embed):
    b, t, _ = x_mark.shape
    xt = jnp.transpose(x_mark, (2, 1, 0))
    idx, c = pl.pallas_call(
        _idx_kernel,
        grid=(b // CHUNK,),
        in_specs=[
            pl.BlockSpec((NFEAT, t, CHUNK), lambda g: (0, 0, g)),
            pl.BlockSpec((NMIN, D), lambda g: (0, 0)),
            pl.BlockSpec((NHOUR, D), lambda g: (0, 0)),
        ],
        out_specs=[
            pl.BlockSpec((1, t, CHUNK), lambda g: (g, 0, 0)),
            pl.BlockSpec((NMIN, NHOUR, D), lambda g: (0, 0, 0)),
        ],
        out_shape=[
            jax.ShapeDtypeStruct((b // CHUNK, t, CHUNK), jnp.int32),
            jax.ShapeDtypeStruct((NMIN, NHOUR, D), jnp.float32),
        ],
    )(xt, minute_embed, hour_embed)
    return idx, c.reshape(NMIN * NHOUR, D)


def _make_gather(n_b, n_t):
    assert n_b == NW * CHUNK
    n_tok = n_b * n_t
    mesh = plsc.VectorSubcoreMesh(
        core_axis_name="c", subcore_axis_name="s", num_cores=NC, num_subcores=NS
    )

    @functools.partial(
        pl.kernel,
        out_type=jax.ShapeDtypeStruct((n_tok, D), jnp.float32),
        mesh=mesh,
        scratch_types=(
            [pltpu.VMEM((n_t, CHUNK), jnp.int32)]
            + [pltpu.VMEM((CHUNK,), jnp.int32) for _ in range(4)]
            + [pltpu.VMEM((CHUNK, D), jnp.float32) for _ in range(4)]
            + [pltpu.SemaphoreType.DMA for _ in range(8)]
            + [pltpu.VMEM_SHARED((NMIN * NHOUR, D), jnp.float32)]
        ),
        compiler_params=pltpu.CompilerParams(needs_layout_passes=False),
    )
    def gather(idx_hbm, c_hbm, out_hbm, slab,
               i0, i1, i2, i3, r0, r1, r2, r3,
               gs0, gs1, gs2, gs3, ss0, ss1, ss2, ss3, c_sp):
        ib = [i0, i1, i2, i3]
        rb = [r0, r1, r2, r3]
        gs = [gs0, gs1, gs2, gs3]
        ss = [ss0, ss1, ss2, ss3]
        wid = lax.axis_index("s") * NC + lax.axis_index("c")
        w_base = wid * n_t * CHUNK

        @pl.when(lax.axis_index("s") == 0)
        def _():
            pltpu.sync_copy(c_hbm, c_sp)

        pltpu.sync_copy(idx_hbm.at[wid], slab)
        plsc.subcore_barrier()

        def fire(ri, ib, rows, gsem, src):
            for jj in range(CHUNK // L):
                q = lax.iota(jnp.int32, L) + (CHUNK * ri + L * jj)
                b_loc = q // n_t
                t = q - b_loc * n_t
                ib[pl.ds(L * jj, L)] = plsc.load_gather(slab, [t, b_loc])
            pltpu.async_copy(src.at[ib], rows, gsem)

        def wait_g(ib, rows, gsem, src):
            pltpu.make_async_copy(src.at[ib], rows, gsem).wait()

        def scatter(ti, rows, ssem):
            pltpu.async_copy(
                rows, out_hbm.at[pl.ds(w_base + ti * CHUNK, CHUNK)], ssem
            )

        def wait_s(ti, rows, ssem):
            pltpu.make_async_copy(
                rows, out_hbm.at[pl.ds(w_base + ti * CHUNK, CHUNK)], ssem
            ).wait()

        n_chunks = n_t
        n_groups = n_chunks // 4
        srcs = [c_sp, c_sp, c_hbm, c_hbm]
        for k in range(3):
            fire(k, ib[k], rb[k], gs[k], srcs[k])

        def body(g, carry):
            c0 = 4 * g
            for k in range(4):
                c = c0 + k
                s3 = (k + 3) % 4
                wait_g(ib[k], rb[k], gs[k], srcs[k])
                scatter(c, rb[k], ss[k])

                @pl.when(c + 3 < n_chunks)
                def _():
                    @pl.when(c >= 1)
                    def _():
                        wait_s(c - 1, rb[s3], ss[s3])

                    fire(c + 3, ib[s3], rb[s3], gs[s3], srcs[s3])

            return carry

        lax.fori_loop(0, n_groups, body, 0)
        for j in range(4):
            c = n_chunks - 4 + j
            wait_s(c, rb[c % 4], ss[c % 4])

    return gather


def kernel(x_mark, minute_embed, hour_embed):
    b, t, _ = x_mark.shape
    idx, c_table = _token_idx(x_mark, minute_embed, hour_embed)
    out = _make_gather(b, t)(idx, c_table)
    return out.reshape(b, t, D)


# speedup vs baseline: 1.2715x; 1.2715x over previous
import functools

import jax
import jax.numpy as jnp
from jax import lax
from jax.experimental import pallas as pl
from jax.experimental.pallas import tpu as pltpu
from jax.experimental.pallas import tpu_sc as plsc

D = 128
NMIN = 60
NHOUR = 24
NC = 2
NS = 16
NW = NC * NS
L = 16
CHUNK = 128
NFEAT = 5
MIN_CH = 3
HOUR_CH = 2


def _idx_kernel(xt_ref, minute_ref, hour_ref, idx_ref, c_ref):
    @pl.when(pl.program_id(0) == 0)
    def _():
        c_ref[...] = minute_ref[...][:, None, :] + hour_ref[...][None, :, :]

    m = (xt_ref[MIN_CH] * 59.0).astype(jnp.int32)
    h = (xt_ref[HOUR_CH] * 23.0).astype(jnp.int32)
    idx_ref[0] = m * NHOUR + h


def _token_idx(x_mark, minute_embed, hour_embed):
    b, t, _ = x_mark.shape
    xt = jnp.transpose(x_mark, (2, 1, 0))
    idx, c = pl.pallas_call(
        _idx_kernel,
        grid=(b // CHUNK,),
        in_specs=[
            pl.BlockSpec((NFEAT, t, CHUNK), lambda g: (0, 0, g)),
            pl.BlockSpec((NMIN, D), lambda g: (0, 0)),
            pl.BlockSpec((NHOUR, D), lambda g: (0, 0)),
        ],
        out_specs=[
            pl.BlockSpec((1, t, CHUNK), lambda g: (g, 0, 0)),
            pl.BlockSpec((NMIN, NHOUR, D), lambda g: (0, 0, 0)),
        ],
        out_shape=[
            jax.ShapeDtypeStruct((b // CHUNK, t, CHUNK), jnp.int32),
            jax.ShapeDtypeStruct((NMIN, NHOUR, D), jnp.float32),
        ],
    )(xt, minute_embed, hour_embed)
    return idx, c.reshape(NMIN * NHOUR, D)


def _make_gather(n_b, n_t):
    assert n_b == NW * CHUNK
    n_tok = n_b * n_t
    mesh = plsc.VectorSubcoreMesh(
        core_axis_name="c", subcore_axis_name="s", num_cores=NC, num_subcores=NS
    )

    @functools.partial(
        pl.kernel,
        out_type=jax.ShapeDtypeStruct((n_tok, D), jnp.float32),
        mesh=mesh,
        scratch_types=(
            [pltpu.VMEM((n_t, CHUNK), jnp.int32)]
            + [pltpu.VMEM((CHUNK,), jnp.int32) for _ in range(4)]
            + [pltpu.VMEM((2 * CHUNK, D), jnp.float32) for _ in range(2)]
            + [pltpu.SemaphoreType.DMA for _ in range(4)]
            + [pltpu.VMEM_SHARED((NMIN * NHOUR, D), jnp.float32)]
        ),
        compiler_params=pltpu.CompilerParams(needs_layout_passes=False),
    )
    def gather(idx_hbm, c_hbm, out_hbm, slab,
               i0, i1, i2, i3, r0, r1, gs0, gs1, ss0, ss1, c_sp):
        ib = [(i0, i1), (i2, i3)]
        rb = [r0, r1]
        gs = [gs0, gs1]
        ss = [ss0, ss1]
        wid = lax.axis_index("s") * NC + lax.axis_index("c")
        w_base = wid * n_t * CHUNK

        @pl.when(lax.axis_index("s") == 0)
        def _():
            pltpu.sync_copy(c_hbm, c_sp)

        pltpu.sync_copy(idx_hbm.at[wid], slab)
        plsc.subcore_barrier()

        def idx_for(ri, ibh):
            for jj in range(CHUNK // L):
                q = lax.iota(jnp.int32, L) + (CHUNK * ri + L * jj)
                b_loc = q // n_t
                t = q - b_loc * n_t
                ibh[pl.ds(L * jj, L)] = plsc.load_gather(slab, [t, b_loc])

        def fire(ui, ibp, rows, gsem):
            idx_for(2 * ui, ibp[0])
            pltpu.async_copy(c_sp.at[ibp[0]], rows.at[pl.ds(0, CHUNK)], gsem)
            idx_for(2 * ui + 1, ibp[1])
            pltpu.async_copy(c_sp.at[ibp[1]], rows.at[pl.ds(CHUNK, CHUNK)], gsem)

        def wait_g(ibp, rows, gsem):
            pltpu.make_async_copy(c_sp.at[ibp[0]], rows.at[pl.ds(0, CHUNK)], gsem).wait()
            pltpu.make_async_copy(c_sp.at[ibp[1]], rows.at[pl.ds(CHUNK, CHUNK)], gsem).wait()

        def scatter(ui, rows, ssem):
            pltpu.async_copy(
                rows, out_hbm.at[pl.ds(w_base + ui * 2 * CHUNK, 2 * CHUNK)], ssem
            )

        def wait_s(ui, rows, ssem):
            pltpu.make_async_copy(
                rows, out_hbm.at[pl.ds(w_base + ui * 2 * CHUNK, 2 * CHUNK)], ssem
            ).wait()

        n_super = n_t // 2
        n_groups = n_super // 2
        fire(0, ib[0], rb[0], gs[0])

        def body(g, carry):
            u0 = 2 * g

            @pl.when(g >= 1)
            def _():
                wait_s(u0 - 1, rb[1], ss[1])

            fire(u0 + 1, ib[1], rb[1], gs[1])
            wait_g(ib[0], rb[0], gs[0])
            scatter(u0, rb[0], ss[0])
            wait_g(ib[1], rb[1], gs[1])
            scatter(u0 + 1, rb[1], ss[1])

            @pl.when(g < n_groups - 1)
            def _():
                wait_s(u0, rb[0], ss[0])
                fire(u0 + 2, ib[0], rb[0], gs[0])

            return carry

        lax.fori_loop(0, n_groups, body, 0)
        wait_s(n_super - 2, rb[0], ss[0])
        wait_s(n_super - 1, rb[1], ss[1])

    return gather


def kernel(x_mark, minute_embed, hour_embed):
    b, t, _ = x_mark.shape
    idx, c_table = _token_idx(x_mark, minute_embed, hour_embed)
    out = _make_gather(b, t)(idx, c_table)
    return out.reshape(b, t, D)


# speedup vs baseline: 1.4035x; 1.1038x over previous
import functools

import jax
import jax.numpy as jnp
from jax import lax
from jax.experimental import pallas as pl
from jax.experimental.pallas import tpu as pltpu
from jax.experimental.pallas import tpu_sc as plsc

D = 128
NMIN = 60
NHOUR = 24
NC = 2
NS = 16
NW = NC * NS
L = 16
CHUNK = 128
NFEAT = 5
MIN_CH = 3
HOUR_CH = 2


def _idx_kernel(xt_ref, minute_ref, hour_ref, idx_ref, c_ref):
    @pl.when(pl.program_id(0) == 0)
    def _():
        c_ref[...] = minute_ref[...][:, None, :] + hour_ref[...][None, :, :]

    m = (xt_ref[MIN_CH] * 59.0).astype(jnp.int32)
    h = (xt_ref[HOUR_CH] * 23.0).astype(jnp.int32)
    idx_ref[0] = m * NHOUR + h


def _token_idx(x_mark, minute_embed, hour_embed):
    b, t, _ = x_mark.shape
    xt = jnp.transpose(x_mark, (2, 1, 0))
    idx, c = pl.pallas_call(
        _idx_kernel,
        grid=(b // CHUNK,),
        in_specs=[
            pl.BlockSpec((NFEAT, t, CHUNK), lambda g: (0, 0, g)),
            pl.BlockSpec((NMIN, D), lambda g: (0, 0)),
            pl.BlockSpec((NHOUR, D), lambda g: (0, 0)),
        ],
        out_specs=[
            pl.BlockSpec((1, t, CHUNK), lambda g: (g, 0, 0)),
            pl.BlockSpec((NMIN, NHOUR, D), lambda g: (0, 0, 0)),
        ],
        out_shape=[
            jax.ShapeDtypeStruct((b // CHUNK, t, CHUNK), jnp.int32),
            jax.ShapeDtypeStruct((NMIN, NHOUR, D), jnp.float32),
        ],
    )(xt, minute_embed, hour_embed)
    return idx, c.reshape(NMIN * NHOUR, D)


def _make_gather(n_b, n_t):
    assert n_b == NW * CHUNK
    n_tok = n_b * n_t
    mesh = plsc.VectorSubcoreMesh(
        core_axis_name="c", subcore_axis_name="s", num_cores=NC, num_subcores=NS
    )

    @functools.partial(
        pl.kernel,
        out_type=jax.ShapeDtypeStruct((n_tok, D), jnp.float32),
        mesh=mesh,
        scratch_types=(
            [pltpu.VMEM((n_t, CHUNK), jnp.int32)]
            + [pltpu.VMEM((CHUNK,), jnp.int32) for _ in range(5)]
            + [pltpu.VMEM((CHUNK, D), jnp.float32) for _ in range(5)]
            + [pltpu.SemaphoreType.DMA for _ in range(10)]
            + [pltpu.VMEM_SHARED((NMIN * NHOUR, D), jnp.float32)]
        ),
        compiler_params=pltpu.CompilerParams(needs_layout_passes=False),
    )
    def gather(idx_hbm, c_hbm, out_hbm, slab,
               i0, i1, i2, i3, i4, r0, r1, r2, r3, r4,
               gs0, gs1, gs2, gs3, gs4, ss0, ss1, ss2, ss3, ss4, c_sp):
        ib = [i0, i1, i2, i3, i4]
        rb = [r0, r1, r2, r3, r4]
        gs = [gs0, gs1, gs2, gs3, gs4]
        ss = [ss0, ss1, ss2, ss3, ss4]
        wid = lax.axis_index("s") * NC + lax.axis_index("c")
        w_base = wid * n_t * CHUNK

        @pl.when(lax.axis_index("s") == 0)
        def _():
            pltpu.sync_copy(c_hbm, c_sp)

        pltpu.sync_copy(idx_hbm.at[wid], slab)
        plsc.subcore_barrier()

        def fire(ri, ib, rows, gsem):
            for jj in range(CHUNK // L):
                q = lax.iota(jnp.int32, L) + (CHUNK * ri + L * jj)
                b_loc = q // n_t
                t = q - b_loc * n_t
                ib[pl.ds(L * jj, L)] = plsc.load_gather(slab, [t, b_loc])
            pltpu.async_copy(c_sp.at[ib], rows, gsem)

        def wait_g(ib, rows, gsem):
            pltpu.make_async_copy(c_sp.at[ib], rows, gsem).wait()

        def scatter(ti, rows, ssem):
            pltpu.async_copy(
                rows, out_hbm.at[pl.ds(w_base + ti * CHUNK, CHUNK)], ssem
            )

        def wait_s(ti, rows, ssem):
            pltpu.make_async_copy(
                rows, out_hbm.at[pl.ds(w_base + ti * CHUNK, CHUNK)], ssem
            ).wait()

        NB = 5
        n_chunks = n_t
        n_groups = n_chunks // NB
        for k in range(NB - 1):
            fire(k, ib[k], rb[k], gs[k])

        def body(g, carry):
            c0 = NB * g
            for k in range(NB):
                c = c0 + k
                s3 = (k + NB - 1) % NB
                wait_g(ib[k], rb[k], gs[k])
                scatter(c, rb[k], ss[k])

                @pl.when(c + NB - 1 < n_chunks)
                def _():
                    @pl.when(c >= 1)
                    def _():
                        wait_s(c - 1, rb[s3], ss[s3])

                    fire(c + NB - 1, ib[s3], rb[s3], gs[s3])

            return carry

        lax.fori_loop(0, n_groups, body, 0)
        for j in range(NB):
            c = n_chunks - NB + j
            wait_s(c, rb[c % NB], ss[c % NB])

    return gather


def kernel(x_mark, minute_embed, hour_embed):
    b, t, _ = x_mark.shape
    idx, c_table = _token_idx(x_mark, minute_embed, hour_embed)
    out = _make_gather(b, t)(idx, c_table)
    return out.reshape(b, t, D)
